# baseline (speedup 1.0000x reference)
import functools

import jax
import jax.numpy as jnp
from jax import lax
from jax.experimental import pallas as pl
from jax.experimental.pallas import tpu as pltpu

DIM = 96
HEADS = 4
H = W = 384
L = H * W
HW4 = L // HEADS
CPH = DIM // HEADS


_HB = 8
_NHB = H // _HB


def _qkv_body(xa_ref, xb_ref, w_ref, dw_ref, out_ref):
    x16 = jnp.concatenate([xa_ref[...], xb_ref[...]], axis=1)
    xh = x16[:, 0:_HB + 2, :]
    mm = jnp.dot(w_ref[...], xh.reshape(DIM, -1),
                 preferred_element_type=jnp.float32,
                 precision=lax.Precision.HIGHEST)
    qh = mm.reshape(DIM, _HB + 2, W + 2)
    acc = jnp.zeros((DIM, _HB, W), dtype=jnp.float32)
    for di in range(3):
        for dj in range(3):
            tap = dw_ref[:, 3 * di + dj][:, None, None]
            acc = acc + qh[:, di:di + _HB, dj:dj + W] * tap
    out_ref[...] = acc


def _dw_body(xa_ref, xb_ref, dw_ref, out_ref):
    x16 = jnp.concatenate([xa_ref[...], xb_ref[...]], axis=1)
    qh = x16[:, 0:_HB + 2, :].astype(jnp.bfloat16).astype(jnp.float32)
    dw = dw_ref[...].astype(jnp.bfloat16).astype(jnp.float32)
    acc = jnp.zeros((DIM, _HB, W), dtype=jnp.float32)
    for di in range(3):
        for dj in range(3):
            tap = dw[:, 3 * di + dj][:, None, None]
            acc = acc + qh[:, di:di + _HB, dj:dj + W] * tap
    out_ref[...] = acc


def _dw_conv(q_pad, dw2d):
    return pl.pallas_call(
        _dw_body,
        grid=(5, _NHB),
        in_specs=[
            pl.BlockSpec((DIM, _HB, W + 2), lambda cb, hb: (cb, hb, 0)),
            pl.BlockSpec((DIM, _HB, W + 2), lambda cb, hb: (cb, hb + 1, 0)),
            pl.BlockSpec((DIM, 9), lambda cb, hb: (cb, 0)),
        ],
        out_specs=pl.BlockSpec((DIM, _HB, W), lambda cb, hb: (cb, hb, 0)),
        out_shape=jax.ShapeDtypeStruct((5 * DIM, H, W), jnp.float32),
    )(q_pad, q_pad, dw2d)


def _qkv_conv(x_pad, w2d, dw2d):
    grid = (5, _NHB)
    return pl.pallas_call(
        _qkv_body,
        grid=grid,
        in_specs=[
            pl.BlockSpec((DIM, _HB, W + 2), lambda cb, hb: (0, hb, 0)),
            pl.BlockSpec((DIM, _HB, W + 2), lambda cb, hb: (0, hb + 1, 0)),
            pl.BlockSpec((DIM, DIM), lambda cb, hb: (cb, 0)),
            pl.BlockSpec((DIM, 9), lambda cb, hb: (cb, 0)),
        ],
        out_specs=pl.BlockSpec((DIM, _HB, W), lambda cb, hb: (cb, hb, 0)),
        out_shape=jax.ShapeDtypeStruct((5 * DIM, H, W), jnp.float32),
    )(x_pad, x_pad, w2d, dw2d)


_PC = 4096
_NPC = HW4 // _PC


def _gram_body(qk_ref, g_ref):
    @pl.when(pl.program_id(1) == 0)
    def _():
        g_ref[...] = jnp.zeros_like(g_ref)
    qk = qk_ref[0]
    g_ref[0] += jnp.dot(qk, qk.T, preferred_element_type=jnp.float32)


def _gram(qk):
    return pl.pallas_call(
        _gram_body,
        grid=(HEADS, _NPC),
        in_specs=[pl.BlockSpec((1, 2 * DIM, _PC), lambda h, p: (h, 0, p))],
        out_specs=pl.BlockSpec((1, 2 * DIM, 2 * DIM), lambda h, p: (h, 0, 0)),
        out_shape=jax.ShapeDtypeStruct((HEADS, 2 * DIM, 2 * DIM), jnp.float32),
    )(qk)


def _attn_body(g_ref, t_ref, a_ref):
    g = g_ref[0]
    n = 2 * DIM
    eye = (lax.broadcasted_iota(jnp.int32, (n, n), 0)
           == lax.broadcasted_iota(jnp.int32, (n, n), 1)).astype(jnp.float32)
    diag = jnp.sum(g * eye, axis=1)
    inv = 1.0 / jnp.maximum(jnp.sqrt(diag), 1e-12)
    sim = g[:DIM, DIM:] * inv[:DIM, None] * inv[None, DIM:]
    t = t_ref[0][0:1, 0:1]
    e = jnp.exp(sim * t)
    a_ref[0] = e / (jnp.sum(e, axis=1, keepdims=True) + 1.0)


def _attn_softmax(g, temp_b):
    return pl.pallas_call(
        _attn_body,
        grid=(HEADS,),
        in_specs=[
            pl.BlockSpec((1, 2 * DIM, 2 * DIM), lambda h: (h, 0, 0)),
            pl.BlockSpec((1, 8, 128), lambda h: (h, 0, 0)),
        ],
        out_specs=pl.BlockSpec((1, DIM, DIM), lambda h: (h, 0, 0)),
        out_shape=jax.ShapeDtypeStruct((HEADS, DIM, DIM), jnp.float32),
    )(g, temp_b)


def _mix_body(a_ref, v_ref, o_ref):
    o_ref[0] = jnp.dot(a_ref[0], v_ref[0], preferred_element_type=jnp.float32)


def _mix(attn, v):
    return pl.pallas_call(
        _mix_body,
        grid=(HEADS, _NPC),
        in_specs=[
            pl.BlockSpec((1, DIM, DIM), lambda h, p: (h, 0, 0)),
            pl.BlockSpec((1, DIM, _PC), lambda h, p: (h, 0, p)),
        ],
        out_specs=pl.BlockSpec((1, DIM, _PC), lambda h, p: (h, 0, p)),
        out_shape=jax.ShapeDtypeStruct((HEADS, DIM, HW4), jnp.float32),
    )(attn, v)


_LC = 8192
_NLC = L // _LC


def _proj_body(w_ref, x_ref, o_ref):
    o_ref[...] = jnp.dot(w_ref[...], x_ref[...],
                         preferred_element_type=jnp.float32)


def _proj(w2d, x2d):
    o, i = w2d.shape
    return pl.pallas_call(
        _proj_body,
        grid=(_NLC,),
        in_specs=[
            pl.BlockSpec((o, i), lambda j: (0, 0)),
            pl.BlockSpec((i, _LC), lambda j: (0, j)),
        ],
        out_specs=pl.BlockSpec((o, _LC), lambda j: (0, j)),
        out_shape=jax.ShapeDtypeStruct((o, L), jnp.float32),
    )(w2d, x2d)


def _scatter_axis(idx, vals, axis):
    grids = list(jnp.indices(idx.shape))
    grids[axis] = idx
    return jnp.zeros_like(vals).at[tuple(grids)].set(vals)


def _fold_box(t):
    return t.reshape(HEADS, CPH, HEADS, HW4).reshape(HEADS, DIM, HW4)


def _fold_nonbox(t):
    t = t.reshape(HEADS, CPH, HW4, HEADS).transpose(0, 1, 3, 2)
    return t.reshape(HEADS, DIM, HW4)


def _unfold_box(t):
    return t.reshape(HEADS, CPH, HEADS, HW4).reshape(DIM, L)


def _unfold_nonbox(t):
    t = t.reshape(HEADS, CPH, HEADS, HW4).transpose(0, 1, 3, 2)
    return t.reshape(DIM, L)


def kernel(x, w_qkv, w_dw, w_out, temperature):
    xs = x[0]
    half = DIM // 2

    xh = xs[:half]
    idx_h = jnp.argsort(xh, axis=-2)
    x_sort = jnp.take_along_axis(xh, idx_h, axis=-2)
    idx_w = jnp.argsort(x_sort, axis=-1)
    x_sort = jnp.take_along_axis(x_sort, idx_w, axis=-1)
    xs = xs.at[:half].set(x_sort)

    _c = _proj(w_qkv[:, :, 0, 0], xs.reshape(DIM, L)).reshape(5 * DIM, H, W)
    qkv = jax.lax.conv_general_dilated(
        _c[None], w_dw, window_strides=(1, 1), padding='SAME',
        feature_group_count=5 * DIM,
        dimension_numbers=('NCHW', 'OIHW', 'NCHW'))[0]
    q1, k1, q2, k2, v = jnp.split(qkv.reshape(5, DIM, L), 5, axis=0)
    q1, k1, q2, k2, v = q1[0], k1[0], q2[0], k2[0], v[0]

    idx = jnp.argsort(v, axis=-1)
    vs = jnp.take_along_axis(v, idx, axis=-1)
    g = lambda t: jnp.take_along_axis(t, idx, axis=-1)
    q1s, k1s, q2s, k2s = g(q1), g(k1), g(q2), g(k2)

    temp_b = jnp.broadcast_to(temperature.reshape(HEADS, 1, 1), (HEADS, 8, 128))

    qk1 = jnp.concatenate([_fold_box(q1s), _fold_box(k1s)], axis=1)
    attn1 = _attn_softmax(_gram(qk1), temp_b)
    out1 = _mix(attn1, _fold_box(vs))

    qk2 = jnp.concatenate([_fold_nonbox(q2s), _fold_nonbox(k2s)], axis=1)
    attn2 = _attn_softmax(_gram(qk2), temp_b)
    out2 = _mix(attn2, _fold_nonbox(vs))

    prod = _unfold_box(out1) * _unfold_nonbox(out2)
    res = _scatter_axis(idx, prod, 1)

    out = _proj(w_out[:, :, 0, 0], res)
    out = out.reshape(DIM, H, W)

    orp = out[:half]
    orp = _scatter_axis(idx_w, orp, 2)
    orp = _scatter_axis(idx_h, orp, 1)
    out = out.at[:half].set(orp)
    return out[None]
